# baseline (speedup 1.0000x reference)
import jax
import jax.numpy as jnp
import numpy as np
from jax import lax
from jax.experimental import pallas as pl
from jax.experimental.pallas import tpu as pltpu
from jax.experimental.pallas import tpu_sc as plsc

_V = 100000
_D = 1024
_GW = 128
_NGF = _V // _GW
_TAIL = _V - _NGF * _GW
_ROWS = 49
_COLS = 2048
_VP = _ROWS * _COLS
_NW = 32
_TCG = 400
_VC = _TCG * _GW
_GPW = -(-(_NGF - _TCG) // _NW)
_RC = 4
_RROWS = _D // _RC
_TOPK = 40
_TOPP = 0.9
_NEG = np.float32(-np.inf)


def _sc_body(src_hbm, seg_hbm, mask_hbm, tok_hbm, segemb_hbm, w_hbm, out_hbm,
             idx_v, sdx_v, trow_v, srow_v, mrow_v, hv, wb0, wb1, ov,
             sem0, sem1, semg):
    wid = lax.axis_index("s") * 2 + lax.axis_index("c")

    pltpu.sync_copy(src_hbm.at[0, pl.ds(2040, 8)], idx_v)
    pltpu.sync_copy(seg_hbm.at[0, pl.ds(2040, 8)], sdx_v)
    pltpu.sync_copy(mask_hbm.at[0, pl.ds(2032, 16)], mrow_v)
    pltpu.async_copy(tok_hbm.at[idx_v], trow_v, semg).wait()
    pltpu.async_copy(segemb_hbm.at[sdx_v], srow_v, semg).wait()
    mb = jnp.broadcast_to(mrow_v[...][15], (16,))
    for k in range(_D // 16):
        hv[pl.ds(k * 16, 16)] = (
            trow_v[7, pl.ds(k * 16, 16)] + srow_v[7, pl.ds(k * 16, 16)]) * mb

    def accum_chunk(rc, wb, accs):
        hbase = rc * _RROWS

        def kd_body(kd, accs):
            hvc = hv[pl.ds(hbase + kd * 16, 16)]
            new = list(accs)
            for j in range(16):
                hb = jnp.broadcast_to(hvc[j], (16,))
                row = kd * 16 + j
                for b in range(8):
                    new[b] = new[b] + hb * wb[row, pl.ds(b * 16, 16)]
            return tuple(new)

        return lax.fori_loop(0, _RROWS // 16, kd_body, accs)

    def group_body(t, carry):
        g = _TCG + wid + t * _NW

        @pl.when(g < _NGF)
        def _():
            c0 = g * _GW
            bufs = (wb0, wb1)
            sems = (sem0, sem1)
            handles = [None] * _RC
            handles[0] = pltpu.async_copy(
                w_hbm.at[pl.ds(0, _RROWS), pl.ds(c0, _GW)], wb0, sem0)
            accs = tuple(jnp.zeros((16,), jnp.float32) for _ in range(8))
            for rc in range(_RC):
                if rc + 1 < _RC:
                    handles[rc + 1] = pltpu.async_copy(
                        w_hbm.at[pl.ds((rc + 1) * _RROWS, _RROWS),
                                 pl.ds(c0, _GW)],
                        bufs[(rc + 1) % 2], sems[(rc + 1) % 2])
                handles[rc].wait()
                accs = accum_chunk(rc, bufs[rc % 2], accs)
            for b in range(8):
                ov[pl.ds(b * 16, 16)] = accs[b]
            pltpu.sync_copy(ov, out_hbm.at[pl.ds(g * _GW, _GW)])

        return carry

    lax.fori_loop(0, _GPW, group_body, 0)


def _sc_matvec(src32, seg32, mask, tok_emb, seg_emb, W):
    mesh = plsc.VectorSubcoreMesh(core_axis_name="c", subcore_axis_name="s")
    f = pl.kernel(
        _sc_body,
        out_type=jax.ShapeDtypeStruct((_VP,), jnp.float32),
        mesh=mesh,
        scratch_types=[
            pltpu.VMEM((8,), jnp.int32),
            pltpu.VMEM((8,), jnp.int32),
            pltpu.VMEM((8, _D), jnp.float32),
            pltpu.VMEM((8, _D), jnp.float32),
            pltpu.VMEM((16,), jnp.float32),
            pltpu.VMEM((_D,), jnp.float32),
            pltpu.VMEM((_RROWS, _GW), jnp.float32),
            pltpu.VMEM((_RROWS, _GW), jnp.float32),
            pltpu.VMEM((_GW,), jnp.float32),
            pltpu.SemaphoreType.DMA,
            pltpu.SemaphoreType.DMA,
            pltpu.SemaphoreType.DMA,
        ],
    )
    return f(src32, seg32, mask, tok_emb, seg_emb, W)


_BD = 32
_NK = _D // _BD
_NQ = 4


def _mv_body(s_ref, tok_ref, seg_ref, m_ref, w_ref, out_ref, h_ref, b0, b1,
             *sems):
    i = pl.program_id(0)
    bufs = (b0, b1)

    def copies(step, bi):
        qr = _BD // _NQ
        return [
            pltpu.make_async_copy(
                w_ref.at[pl.ds(step * _BD + q * qr, qr), pl.ds(0, _VC)],
                bufs[bi].at[pl.ds(q * qr, qr), :],
                sems[bi * _NQ + q])
            for q in range(_NQ)
        ]

    @pl.when(i == 0)
    def _init():
        h = (tok_ref[0] + seg_ref[0]) * m_ref[...]
        h_ref[...] = jnp.transpose(h, (1, 0))
        for c in copies(0, 0):
            c.start()
        for c in copies(1, 1):
            c.start()

    def do_step(bi):
        for c in copies(i, bi):
            c.wait()
        hs = h_ref[pl.ds(i * _BD, _BD), :]
        part = jax.lax.dot_general(
            hs, bufs[bi][...], (((0,), (0,)), ((), ())),
            preferred_element_type=jnp.float32)

        @pl.when(i == 0)
        def _first():
            out_ref[...] = part

        @pl.when(i > 0)
        def _rest():
            out_ref[...] = out_ref[...] + part

        @pl.when(i + 2 < _NK)
        def _prefetch():
            for c in copies(i + 2, bi):
                c.start()

    par = lax.rem(i, 2)

    @pl.when(par == 0)
    def _even():
        do_step(0)

    @pl.when(par == 1)
    def _odd():
        do_step(1)


def _tc_matvec(idxs, tok_emb, seg_emb, mlast, W):
    grid_spec = pltpu.PrefetchScalarGridSpec(
        num_scalar_prefetch=1,
        grid=(_NK,),
        in_specs=[
            pl.BlockSpec((1, 1, _D), lambda i, s: (s[0], 0, 0)),
            pl.BlockSpec((1, 1, _D), lambda i, s: (s[1], 0, 0)),
            pl.BlockSpec((1, 1), lambda i, s: (0, 0)),
            pl.BlockSpec(memory_space=pltpu.MemorySpace.HBM),
        ],
        out_specs=pl.BlockSpec((1, _VC), lambda i, s: (0, 0)),
        scratch_shapes=[
            pltpu.VMEM((_D, 1), jnp.float32),
            pltpu.VMEM((_BD, _VC), jnp.float32),
            pltpu.VMEM((_BD, _VC), jnp.float32),
        ] + [pltpu.SemaphoreType.DMA] * (2 * _NQ),
    )
    return pl.pallas_call(
        _mv_body,
        grid_spec=grid_spec,
        out_shape=jax.ShapeDtypeStruct((1, _VC), jnp.float32),
    )(idxs, tok_emb.reshape(_V, 1, _D), seg_emb.reshape(2, 1, _D), mlast, W)


def _sel_body(s_ref, tok_ref, seg_ref, m_ref, wt_ref, ltc_ref, l_ref, g_ref,
              out_ref, lm_ref):
    ridx = jax.lax.broadcasted_iota(jnp.int32, (_ROWS, _COLS), 0)
    cidx = jax.lax.broadcasted_iota(jnp.int32, (_ROWS, _COLS), 1)
    fidx = ridx * _COLS + cidx
    lm_ref[...] = jnp.where(fidx < _NGF * _GW, l_ref[...], _NEG)
    for r in range(_VC // _COLS):
        lm_ref[r:r + 1, :] = ltc_ref[0:1, r * _COLS:(r + 1) * _COLS]
    h = (tok_ref[0] + seg_ref[0]) * m_ref[...]
    hrep = jnp.broadcast_to(h, (8, _D))
    tail = jax.lax.dot_general(
        hrep, wt_ref[...], (((1,), (0,)), ((), ())),
        preferred_element_type=jnp.float32)
    tcol = jax.lax.broadcasted_iota(jnp.int32, (1, _GW), 1)
    trow = _NGF * _GW // _COLS
    tcoff = _NGF * _GW % _COLS
    lm_ref[trow:trow + 1, tcoff:tcoff + _GW] = jnp.where(
        tcol < _TAIL, tail[0:1, :], _NEG)

    lane = jax.lax.broadcasted_iota(jnp.int32, (1, 128), 1)

    def step(t, carry):
        m_prev, vals, counts = carry
        Lv = lm_ref[...]
        m = jnp.max(jnp.where(Lv < m_prev, Lv, _NEG))
        c = jnp.sum(jnp.where(Lv == m, 1.0, 0.0).astype(jnp.float32))
        vals = jnp.where(lane == t, m, vals)
        counts = jnp.where(lane == t, c, counts)
        return m, vals, counts

    _, vals, counts = jax.lax.fori_loop(
        0, _TOPK, step,
        (np.float32(np.inf),
         jnp.full((1, 128), _NEG, jnp.float32),
         jnp.zeros((1, 128), jnp.float32)))

    mtop = jnp.max(vals)
    tri = (jax.lax.broadcasted_iota(jnp.int32, (128, 128), 0)
           <= jax.lax.broadcasted_iota(jnp.int32, (128, 128), 1)
           ).astype(jnp.float32)
    cum_counts = jax.lax.dot_general(
        counts, tri, (((1,), (0,)), ((), ())),
        precision=jax.lax.Precision.HIGHEST)
    excl = cum_counts - counts
    kept = jnp.logical_and(excl < np.float32(_TOPK), counts > 0.0)
    p_raw = jnp.exp(vals - mtop)
    w = jnp.where(kept, counts * p_raw, 0.0)
    Z = jnp.sum(w)
    p = p_raw / Z
    Cw = jax.lax.dot_general(
        w, tri, (((1,), (0,)), ((), ())),
        precision=jax.lax.Precision.HIGHEST) / Z
    cond = jnp.logical_and(kept, (Cw - p) > np.float32(_TOPP))
    kth = jnp.min(jnp.where(kept, vals, np.float32(np.inf)))
    cutoff = jnp.maximum(jnp.max(jnp.where(cond, vals, _NEG)), kth)

    L = lm_ref[...]
    Y = jnp.where(L >= cutoff, L + g_ref[...], _NEG)
    ymax = jnp.max(Y)
    widx = jnp.min(jnp.where(Y == ymax, fidx, np.int32(2**31 - 1)))
    out_ref[...] = jnp.broadcast_to(widx, (1, 1))


def _tc_select(idxs, tok_emb, seg_emb, mlast, W, Ltc, L2, g2):
    grid_spec = pltpu.PrefetchScalarGridSpec(
        num_scalar_prefetch=1,
        grid=(1,),
        in_specs=[
            pl.BlockSpec((1, 1, _D), lambda i, s: (s[0], 0, 0)),
            pl.BlockSpec((1, 1, _D), lambda i, s: (s[1], 0, 0)),
            pl.BlockSpec((1, 1), lambda i, s: (0, 0)),
            pl.BlockSpec((_D, _GW), lambda i, s: (0, _NGF)),
            pl.BlockSpec((1, _VC), lambda i, s: (0, 0)),
            pl.BlockSpec((_ROWS, _COLS), lambda i, s: (0, 0)),
            pl.BlockSpec((_ROWS, _COLS), lambda i, s: (0, 0)),
        ],
        out_specs=pl.BlockSpec((1, 1), lambda i, s: (0, 0)),
        scratch_shapes=[pltpu.VMEM((_ROWS, _COLS), jnp.float32)],
    )
    return pl.pallas_call(
        _sel_body,
        grid_spec=grid_spec,
        out_shape=jax.ShapeDtypeStruct((1, 1), jnp.int32),
    )(idxs, tok_emb.reshape(_V, 1, _D), seg_emb.reshape(2, 1, _D),
      mlast, W, Ltc, L2, g2)


def kernel(src_tensor, seg_tensor, mask, tok_emb, seg_emb, W):
    src32 = src_tensor.astype(jnp.int32)
    seg32 = seg_tensor.astype(jnp.int32)
    L2 = _sc_matvec(src32, seg32, mask, tok_emb, seg_emb, W).reshape(
        _ROWS, _COLS)
    g = jax.random.gumbel(jax.random.key(42), (1, _V), jnp.float32)
    g2 = jnp.pad(g, ((0, 0), (0, _VP - _V))).reshape(_ROWS, _COLS)
    idxs = jnp.concatenate(
        [src_tensor[0, -1:], seg_tensor[0, -1:]]).astype(jnp.int32)
    mlast = mask[:, -1:]
    Ltc = _tc_matvec(idxs, tok_emb, seg_emb, mlast, W)
    return _tc_select(idxs, tok_emb, seg_emb, mlast, W, Ltc, L2, g2)


# speedup vs baseline: 1.0345x; 1.0345x over previous
import jax
import jax.numpy as jnp
import numpy as np
from jax import lax
from jax.experimental import pallas as pl
from jax.experimental.pallas import tpu as pltpu
from jax.experimental.pallas import tpu_sc as plsc

_V = 100000
_D = 1024
_GW = 128
_NGF = _V // _GW
_TAIL = _V - _NGF * _GW
_ROWS = 49
_COLS = 2048
_VP = _ROWS * _COLS
_NW = 32
_GPW = -(-_NGF // _NW)
_RC = 4
_RROWS = _D // _RC
_TOPK = 40
_TOPP = 0.9
_NEG = np.float32(-np.inf)


def _sc_body(src_hbm, seg_hbm, mask_hbm, tok_hbm, segemb_hbm, w_hbm, out_hbm,
             idx_v, sdx_v, trow_v, srow_v, mrow_v, hv, wb0, wb1, ov,
             sem0, sem1, semg):
    wid = lax.axis_index("s") * 2 + lax.axis_index("c")

    pltpu.sync_copy(src_hbm.at[0, pl.ds(2040, 8)], idx_v)
    pltpu.sync_copy(seg_hbm.at[0, pl.ds(2040, 8)], sdx_v)
    pltpu.sync_copy(mask_hbm.at[0, pl.ds(2032, 16)], mrow_v)
    pltpu.async_copy(tok_hbm.at[idx_v], trow_v, semg).wait()
    pltpu.async_copy(segemb_hbm.at[sdx_v], srow_v, semg).wait()
    mb = jnp.broadcast_to(mrow_v[...][15], (16,))
    for k in range(_D // 16):
        hv[pl.ds(k * 16, 16)] = (
            trow_v[7, pl.ds(k * 16, 16)] + srow_v[7, pl.ds(k * 16, 16)]) * mb

    def accum_chunk(rc, wb, accs):
        hbase = rc * _RROWS

        def kd_body(kd, accs):
            hvc = hv[pl.ds(hbase + kd * 16, 16)]
            new = list(accs)
            for j in range(16):
                hb = jnp.broadcast_to(hvc[j], (16,))
                row = kd * 16 + j
                for b in range(8):
                    new[b] = new[b] + hb * wb[row, pl.ds(b * 16, 16)]
            return tuple(new)

        return lax.fori_loop(0, _RROWS // 16, kd_body, accs)

    def group_body(t, carry):
        g = wid + t * _NW

        @pl.when(g < _NGF)
        def _():
            c0 = g * _GW
            bufs = (wb0, wb1)
            sems = (sem0, sem1)
            handles = [None] * _RC
            handles[0] = pltpu.async_copy(
                w_hbm.at[pl.ds(0, _RROWS), pl.ds(c0, _GW)], wb0, sem0)
            accs = tuple(jnp.zeros((16,), jnp.float32) for _ in range(8))
            for rc in range(_RC):
                if rc + 1 < _RC:
                    handles[rc + 1] = pltpu.async_copy(
                        w_hbm.at[pl.ds((rc + 1) * _RROWS, _RROWS),
                                 pl.ds(c0, _GW)],
                        bufs[(rc + 1) % 2], sems[(rc + 1) % 2])
                handles[rc].wait()
                accs = accum_chunk(rc, bufs[rc % 2], accs)
            for b in range(8):
                ov[pl.ds(b * 16, 16)] = accs[b]
            pltpu.sync_copy(ov, out_hbm.at[pl.ds(g * _GW, _GW)])

        return carry

    lax.fori_loop(0, _GPW, group_body, 0)


def _sc_matvec(src32, seg32, mask, tok_emb, seg_emb, W):
    mesh = plsc.VectorSubcoreMesh(core_axis_name="c", subcore_axis_name="s")
    f = pl.kernel(
        _sc_body,
        out_type=jax.ShapeDtypeStruct((_VP,), jnp.float32),
        mesh=mesh,
        scratch_types=[
            pltpu.VMEM((8,), jnp.int32),
            pltpu.VMEM((8,), jnp.int32),
            pltpu.VMEM((8, _D), jnp.float32),
            pltpu.VMEM((8, _D), jnp.float32),
            pltpu.VMEM((16,), jnp.float32),
            pltpu.VMEM((_D,), jnp.float32),
            pltpu.VMEM((_RROWS, _GW), jnp.float32),
            pltpu.VMEM((_RROWS, _GW), jnp.float32),
            pltpu.VMEM((_GW,), jnp.float32),
            pltpu.SemaphoreType.DMA,
            pltpu.SemaphoreType.DMA,
            pltpu.SemaphoreType.DMA,
        ],
    )
    return f(src32, seg32, mask, tok_emb, seg_emb, W)


def _sel_body(s_ref, tok_ref, seg_ref, m_ref, wt_ref, l_ref, g_ref,
              out_ref, lm_ref):
    ridx = jax.lax.broadcasted_iota(jnp.int32, (_ROWS, _COLS), 0)
    cidx = jax.lax.broadcasted_iota(jnp.int32, (_ROWS, _COLS), 1)
    fidx = ridx * _COLS + cidx
    lm_ref[...] = jnp.where(fidx < _NGF * _GW, l_ref[...], _NEG)
    h = (tok_ref[0] + seg_ref[0]) * m_ref[...]
    hrep = jnp.broadcast_to(h, (8, _D))
    tail = jax.lax.dot_general(
        hrep, wt_ref[...], (((1,), (0,)), ((), ())),
        preferred_element_type=jnp.float32)
    tcol = jax.lax.broadcasted_iota(jnp.int32, (1, _GW), 1)
    trow = _NGF * _GW // _COLS
    tcoff = _NGF * _GW % _COLS
    lm_ref[trow:trow + 1, tcoff:tcoff + _GW] = jnp.where(
        tcol < _TAIL, tail[0:1, :], _NEG)

    lane = jax.lax.broadcasted_iota(jnp.int32, (1, 128), 1)

    def step(t, carry):
        m_prev, vals, counts = carry
        Lv = lm_ref[...]
        m = jnp.max(jnp.where(Lv < m_prev, Lv, _NEG))
        c = jnp.sum(jnp.where(Lv == m, 1.0, 0.0).astype(jnp.float32))
        vals = jnp.where(lane == t, m, vals)
        counts = jnp.where(lane == t, c, counts)
        return m, vals, counts

    _, vals, counts = jax.lax.fori_loop(
        0, _TOPK, step,
        (np.float32(np.inf),
         jnp.full((1, 128), _NEG, jnp.float32),
         jnp.zeros((1, 128), jnp.float32)))

    mtop = jnp.max(vals)
    tri = (jax.lax.broadcasted_iota(jnp.int32, (128, 128), 0)
           <= jax.lax.broadcasted_iota(jnp.int32, (128, 128), 1)
           ).astype(jnp.float32)
    cum_counts = jax.lax.dot_general(
        counts, tri, (((1,), (0,)), ((), ())),
        precision=jax.lax.Precision.HIGHEST)
    excl = cum_counts - counts
    kept = jnp.logical_and(excl < np.float32(_TOPK), counts > 0.0)
    p_raw = jnp.exp(vals - mtop)
    w = jnp.where(kept, counts * p_raw, 0.0)
    Z = jnp.sum(w)
    p = p_raw / Z
    Cw = jax.lax.dot_general(
        w, tri, (((1,), (0,)), ((), ())),
        precision=jax.lax.Precision.HIGHEST) / Z
    cond = jnp.logical_and(kept, (Cw - p) > np.float32(_TOPP))
    kth = jnp.min(jnp.where(kept, vals, np.float32(np.inf)))
    cutoff = jnp.maximum(jnp.max(jnp.where(cond, vals, _NEG)), kth)

    L = lm_ref[...]
    Y = jnp.where(L >= cutoff, L + g_ref[...], _NEG)
    ymax = jnp.max(Y)
    widx = jnp.min(jnp.where(Y == ymax, fidx, np.int32(2**31 - 1)))
    out_ref[...] = jnp.broadcast_to(widx, (1, 1))


def _tc_select(idxs, tok_emb, seg_emb, mlast, W, L2, g2):
    grid_spec = pltpu.PrefetchScalarGridSpec(
        num_scalar_prefetch=1,
        grid=(1,),
        in_specs=[
            pl.BlockSpec((1, 1, _D), lambda i, s: (s[0], 0, 0)),
            pl.BlockSpec((1, 1, _D), lambda i, s: (s[1], 0, 0)),
            pl.BlockSpec((1, 1), lambda i, s: (0, 0)),
            pl.BlockSpec((_D, _GW), lambda i, s: (0, _NGF)),
            pl.BlockSpec((_ROWS, _COLS), lambda i, s: (0, 0)),
            pl.BlockSpec((_ROWS, _COLS), lambda i, s: (0, 0)),
        ],
        out_specs=pl.BlockSpec((1, 1), lambda i, s: (0, 0)),
        scratch_shapes=[pltpu.VMEM((_ROWS, _COLS), jnp.float32)],
    )
    return pl.pallas_call(
        _sel_body,
        grid_spec=grid_spec,
        out_shape=jax.ShapeDtypeStruct((1, 1), jnp.int32),
    )(idxs, tok_emb.reshape(_V, 1, _D), seg_emb.reshape(2, 1, _D),
      mlast, W, L2, g2)


def kernel(src_tensor, seg_tensor, mask, tok_emb, seg_emb, W):
    src32 = src_tensor.astype(jnp.int32)
    seg32 = seg_tensor.astype(jnp.int32)
    L2 = _sc_matvec(src32, seg32, mask, tok_emb, seg_emb, W).reshape(
        _ROWS, _COLS)
    g = jax.random.gumbel(jax.random.key(42), (1, _V), jnp.float32)
    g2 = jnp.pad(g, ((0, 0), (0, _VP - _V))).reshape(_ROWS, _COLS)
    idxs = jnp.concatenate(
        [src_tensor[0, -1:], seg_tensor[0, -1:]]).astype(jnp.int32)
    mlast = mask[:, -1:]
    return _tc_select(idxs, tok_emb, seg_emb, mlast, W, L2, g2)
